# baseline (speedup 1.0000x reference)
import functools

import jax
import jax.numpy as jnp
from jax import lax
from jax.experimental import pallas as pl
from jax.experimental.pallas import tpu as pltpu
from jax.experimental.pallas import tpu_sc as plsc

N = 10000
E = 320000
D_IN = 128
D_OUT = 4

NC = 2
NS = 16
NW = NC * NS
CHUNK = 64
NBUF = 8
CPW = 160
NGRP = CPW // NBUF
E_PAD = NW * CPW * CHUNK
DUMMY = N
N_PAD = ((N + NS * 8 - 1) // (NS * 8)) * (NS * 8)
RPB = N_PAD // NS

_mesh = plsc.VectorSubcoreMesh(
    core_axis_name="c", subcore_axis_name="s", num_cores=NC, num_subcores=NS
)


@functools.partial(
    pl.kernel,
    out_type=jax.ShapeDtypeStruct((NC, N_PAD, D_OUT), jnp.float32),
    mesh=_mesh,
    scratch_types=[
        pltpu.VMEM((CPW, CHUNK), jnp.int32),
        pltpu.VMEM((CPW, CHUNK), jnp.int32),
        pltpu.VMEM((NBUF, CHUNK, D_OUT), jnp.float32),
        pltpu.VMEM((N_PAD, D_OUT), jnp.float32),
        pltpu.VMEM_SHARED((N_PAD, D_OUT), jnp.float32),
        pltpu.SemaphoreType.DMA,
        pltpu.SemaphoreType.DMA,
    ],
    compiler_params=pltpu.CompilerParams(
        use_tc_tiling_on_sc=False, needs_layout_passes=False),
)
def _prop(table_hbm, init_hbm, rows_hbm, cols_hbm, out_hbm,
          rowv, colv, msgs, tbl, acc, gsem, ssem):
    c = lax.axis_index("c")
    s = lax.axis_index("s")
    w = c * NS + s
    pltpu.sync_copy(rows_hbm.at[w], rowv)
    pltpu.sync_copy(cols_hbm.at[w], colv)
    pltpu.sync_copy(table_hbm, tbl)
    pltpu.sync_copy(init_hbm.at[pl.ds(s * RPB, RPB)],
                    acc.at[pl.ds(s * RPB, RPB)])
    plsc.subcore_barrier()

    lanes = lax.iota(jnp.int32, 16)

    def group(g, carry):
        sds = []
        for b in range(NBUF):
            j = g * NBUF + b
            for i in range(CHUNK // 16):
                rows16 = rowv[j, pl.ds(i * 16, 16)]
                pos = lanes + (i * 16)
                for d in range(D_OUT):
                    dd = jnp.full((16,), d, jnp.int32)
                    v = plsc.load_gather(tbl, [rows16, dd])
                    plsc.store_scatter(msgs.at[b], [pos, dd], v)
            sds.append(pltpu.async_copy(msgs.at[b],
                                        acc.at[colv.at[j]],
                                        ssem, add=True))
        for dsc in sds:
            dsc.wait()
        return carry

    lax.fori_loop(0, NGRP, group, 0)
    plsc.subcore_barrier()
    pltpu.sync_copy(acc.at[pl.ds(s * RPB, RPB)],
                    out_hbm.at[c, pl.ds(s * RPB, RPB)])


def _tc_prep(x_ref, w_ref, degp_ref, u_ref, dis_ref, dinv_ref):
    z = jnp.dot(x_ref[...], w_ref[...], preferred_element_type=jnp.float32)
    deg = degp_ref[0] + degp_ref[1] + 1.0
    dis = lax.rsqrt(deg)
    u_ref[...] = dis * z
    dis_ref[...] = dis
    dinv_ref[...] = 1.0 / deg


def _tc_mid(p_ref, u_ref, dinv_ref, w_ref):
    v = p_ref[0] + p_ref[1] - u_ref[...]
    w_ref[...] = v * dinv_ref[...]


def _tc_final(q_ref, w_ref, dis_ref, b_ref, out_ref):
    t = q_ref[0] + q_ref[1] - w_ref[...]
    o = dis_ref[...] * t + b_ref[...]
    m = jnp.max(o, axis=1, keepdims=True)
    e = jnp.exp(o - m)
    lse = jnp.log(jnp.sum(e, axis=1, keepdims=True))
    out_ref[...] = o - m - lse


def kernel(x, edge_index, W, b):
    f32 = jnp.float32
    rows = edge_index[0]
    cols = edge_index[1]
    pad = E_PAD - E
    rows3 = jnp.concatenate(
        [rows, jnp.zeros((pad,), jnp.int32)]).reshape(NW, CPW, CHUNK)
    cols3 = jnp.concatenate(
        [cols, jnp.full((pad,), DUMMY, jnp.int32)]).reshape(NW, CPW, CHUNK)
    x_pad = jnp.pad(x, ((0, N_PAD - N), (0, 0)))
    ones_t = jnp.ones((N_PAD, D_OUT), f32)
    zeros_t = jnp.zeros((N_PAD, D_OUT), f32)

    degp = _prop(ones_t, zeros_t, rows3, cols3)

    u, dis, dinv = pl.pallas_call(
        _tc_prep,
        out_shape=[jax.ShapeDtypeStruct((N_PAD, D_OUT), f32)] * 3,
    )(x_pad, W, degp)

    p = _prop(u, u, rows3, cols3)

    w = pl.pallas_call(
        _tc_mid,
        out_shape=jax.ShapeDtypeStruct((N_PAD, D_OUT), f32),
    )(p, u, dinv)

    q = _prop(w, w, rows3, cols3)

    out = pl.pallas_call(
        _tc_final,
        out_shape=jax.ShapeDtypeStruct((N_PAD, D_OUT), f32),
    )(q, w, dis, b)

    return out[:N]


# speedup vs baseline: 1.1601x; 1.1601x over previous
import functools

import jax
import jax.numpy as jnp
from jax import lax
from jax.experimental import pallas as pl
from jax.experimental.pallas import tpu as pltpu
from jax.experimental.pallas import tpu_sc as plsc

N = 10000
E = 320000
D_IN = 128
D_OUT = 4

NC = 2
DC = 2
NS = 16
EPC = 64
NBUF = 8
CPT = -(-E // (NS * EPC))
CPT = -(-CPT // NBUF) * NBUF
NGRP = CPT // NBUF
E_PAD = NS * CPT * EPC
DUMMY = N
N_PAD = 10240
RPB = N_PAD // NS
FPB = RPB * DC
VE = FPB // 16

_mesh = plsc.VectorSubcoreMesh(
    core_axis_name="c", subcore_axis_name="s", num_cores=NC, num_subcores=NS
)


@functools.partial(
    pl.kernel,
    out_type=jax.ShapeDtypeStruct((NC, N_PAD * DC), jnp.float32),
    mesh=_mesh,
    scratch_types=[
        pltpu.VMEM((CPT, EPC), jnp.int32),
        pltpu.VMEM((CPT, 2 * EPC), jnp.int32),
        pltpu.VMEM((NBUF, 2 * EPC), jnp.float32),
        pltpu.VMEM((N_PAD * DC,), jnp.float32),
        pltpu.VMEM((2 * EPC,), jnp.float32),
        pltpu.VMEM((FPB,), jnp.float32),
        pltpu.VMEM((FPB,), jnp.float32),
        pltpu.VMEM((FPB,), jnp.float32),
        pltpu.VMEM((FPB,), jnp.float32),
        pltpu.VMEM_SHARED((N_PAD * DC,), jnp.float32),
        pltpu.VMEM_SHARED((N_PAD * DC,), jnp.float32),
        pltpu.SemaphoreType.DMA,
        pltpu.SemaphoreType.DMA,
    ],
    compiler_params=pltpu.CompilerParams(
        use_tc_tiling_on_sc=False, needs_layout_passes=False),
)
def _sgconv_sc(z_hbm, rows_hbm, colx_hbm, out_hbm,
               rowv, colx, msgs, tbl, ones, zb, disb, dinvb, tmpb,
               acc, tsh, zsem, ssem):
    c = lax.axis_index("c")
    s = lax.axis_index("s")
    fsl = pl.ds(s * FPB, FPB)

    zcp = pltpu.async_copy(z_hbm.at[c, fsl], zb, zsem)
    pltpu.sync_copy(rows_hbm.at[s], rowv)
    pltpu.sync_copy(colx_hbm.at[s], colx)

    lanes = lax.iota(jnp.int32, 16)
    half = jnp.full((16,), 0.5, jnp.float32)
    three_half = jnp.full((16,), 1.5, jnp.float32)
    magic = jnp.full((16,), 0x5F3759DF, jnp.int32)

    def fill_ones(i, carry):
        ones[pl.ds(i * 16, 16)] = jnp.full((16,), 1.0, jnp.float32)
        return carry

    def fill_zero(i, carry):
        tmpb[pl.ds(i * 16, 16)] = jnp.zeros((16,), jnp.float32)
        return carry

    lax.fori_loop(0, 2 * EPC // 16, fill_ones, 0)
    lax.fori_loop(0, VE, fill_zero, 0)
    pltpu.sync_copy(tmpb, acc.at[fsl])
    plsc.subcore_barrier()

    def deg_group(g, carry):
        sds = [pltpu.async_copy(ones, acc.at[colx.at[g * NBUF + b]],
                                ssem, add=True)
               for b in range(NBUF)]
        for d in sds:
            d.wait()
        return carry

    lax.fori_loop(0, NGRP, deg_group, 0)
    plsc.subcore_barrier()

    pltpu.sync_copy(acc.at[fsl], tmpb)
    zcp.wait()

    def newton(i, carry):
        ix = pl.ds(i * 16, 16)
        d16 = tmpb[ix] + 1.0
        h = d16 * half
        yi = magic - lax.shift_right_logical(plsc.bitcast(d16, jnp.int32), 1)
        y = plsc.bitcast(yi, jnp.float32)
        y = y * (three_half - h * y * y)
        y = y * (three_half - h * y * y)
        y = y * (three_half - h * y * y)
        disb[ix] = y
        dinvb[ix] = y * y
        zb[ix] = y * zb[ix]
        return carry

    lax.fori_loop(0, VE, newton, 0)
    pltpu.sync_copy(zb, acc.at[fsl])
    pltpu.sync_copy(zb, tsh.at[fsl])
    plsc.subcore_barrier()
    pltpu.sync_copy(tsh, tbl)

    def hop():
        def group(g, carry):
            sds = []
            for b in range(NBUF):
                j = g * NBUF + b
                for d in range(DC):
                    for i in range(EPC // 16):
                        rows16 = rowv[j, pl.ds(i * 16, 16)]
                        v = plsc.load_gather(tbl, [rows16 * 2 + d])
                        msgs[b, pl.ds(d * EPC + i * 16, 16)] = v
                sds.append(pltpu.async_copy(msgs.at[b],
                                            acc.at[colx.at[j]],
                                            ssem, add=True))
            for dsc in sds:
                dsc.wait()
            return carry

        lax.fori_loop(0, NGRP, group, 0)
        plsc.subcore_barrier()

    hop()

    def scale_w(i, carry):
        ix = pl.ds(i * 16, 16)
        tmpb[ix] = tmpb[ix] * dinvb[ix]
        return carry

    pltpu.sync_copy(acc.at[fsl], tmpb)
    lax.fori_loop(0, VE, scale_w, 0)
    pltpu.sync_copy(tmpb, acc.at[fsl])
    pltpu.sync_copy(tmpb, tsh.at[fsl])
    plsc.subcore_barrier()
    pltpu.sync_copy(tsh, tbl)

    hop()

    def scale_out(i, carry):
        ix = pl.ds(i * 16, 16)
        tmpb[ix] = tmpb[ix] * disb[ix]
        return carry

    pltpu.sync_copy(acc.at[fsl], tmpb)
    lax.fori_loop(0, VE, scale_out, 0)
    pltpu.sync_copy(tmpb, out_hbm.at[c, fsl])


def _tc_z(x_ref, w_ref, z_ref):
    z_ref[...] = jnp.dot(x_ref[...], w_ref[...],
                         preferred_element_type=jnp.float32)


def _tc_final(h_ref, b_ref, out_ref):
    o = h_ref[...] + b_ref[...]
    m = jnp.max(o, axis=1, keepdims=True)
    e = jnp.exp(o - m)
    lse = jnp.log(jnp.sum(e, axis=1, keepdims=True))
    out_ref[...] = o - m - lse


def kernel(x, edge_index, W, b):
    f32 = jnp.float32
    rows = edge_index[0]
    cols = edge_index[1]
    pad = E_PAD - E
    rows_p = jnp.concatenate([rows, jnp.zeros((pad,), jnp.int32)])
    cols_p = jnp.concatenate([cols, jnp.full((pad,), DUMMY, jnp.int32)])
    rows3 = rows_p.reshape(NS, CPT, EPC)
    cc = cols_p.reshape(NS, CPT, EPC)
    colx3 = jnp.concatenate([cc * 2, cc * 2 + 1], axis=2)

    x_pad = jnp.pad(x, ((0, N_PAD - N), (0, 0)))
    z4 = pl.pallas_call(
        _tc_z,
        out_shape=jax.ShapeDtypeStruct((N_PAD, D_OUT), f32),
    )(x_pad, W)
    zs = jnp.transpose(z4.reshape(N_PAD, NC, DC), (1, 0, 2)).reshape(
        NC, N_PAD * DC)

    h = _sgconv_sc(zs, rows3, colx3)
    h4 = jnp.transpose(h.reshape(NC, N_PAD, DC), (1, 0, 2)).reshape(
        N_PAD, D_OUT)

    out = pl.pallas_call(
        _tc_final,
        out_shape=jax.ShapeDtypeStruct((N_PAD, D_OUT), f32),
    )(h4, b)
    return out[:N]


# speedup vs baseline: 1.3589x; 1.1714x over previous
import functools

import jax
import jax.numpy as jnp
from jax import lax
from jax.experimental import pallas as pl
from jax.experimental.pallas import tpu as pltpu
from jax.experimental.pallas import tpu_sc as plsc

N = 10000
E = 320000
D_IN = 128
D_OUT = 4

NC = 2
DC = 2
NS = 16
UNROLL = 4
EPT = -(-E // (NS * 16 * UNROLL)) * 16 * UNROLL
NBLK = EPT // (16 * UNROLL)
E_PAD = NS * EPT
DUMMY = N
N_PAD = 10240
RPB = N_PAD // NS
FPB = RPB * DC
VE = FPB // 16
FW = N_PAD * DC

_mesh = plsc.VectorSubcoreMesh(
    core_axis_name="c", subcore_axis_name="s", num_cores=NC, num_subcores=NS
)


@functools.partial(
    pl.kernel,
    out_type=jax.ShapeDtypeStruct((NC, FW), jnp.float32),
    mesh=_mesh,
    scratch_types=[
        pltpu.VMEM((EPT,), jnp.int32),
        pltpu.VMEM((EPT,), jnp.int32),
        pltpu.VMEM((FW,), jnp.float32),
        pltpu.VMEM((FW,), jnp.float32),
        pltpu.VMEM((FPB,), jnp.float32),
        pltpu.VMEM((FPB,), jnp.float32),
        pltpu.VMEM((FPB,), jnp.float32),
        pltpu.VMEM((FPB,), jnp.float32),
        pltpu.VMEM((NS, FPB), jnp.float32),
        pltpu.VMEM_SHARED((NS, FW), jnp.float32),
        pltpu.VMEM_SHARED((FW,), jnp.float32),
        pltpu.SemaphoreType.DMA,
        pltpu.SemaphoreType.DMA,
    ],
    compiler_params=pltpu.CompilerParams(
        use_tc_tiling_on_sc=False, needs_layout_passes=False),
)
def _sgconv_sc(z_hbm, rows_hbm, cols_hbm, out_hbm,
               rowv, colv, tbl, acct, zb, disb, dinvb, tmpb, tmp16,
               psh, tsh, zsem, rsem):
    c = lax.axis_index("c")
    s = lax.axis_index("s")
    fsl = pl.ds(s * FPB, FPB)

    zcp = pltpu.async_copy(z_hbm.at[c, fsl], zb, zsem)
    pltpu.sync_copy(rows_hbm.at[s], rowv)
    pltpu.sync_copy(cols_hbm.at[s], colv)

    half = jnp.full((16,), 0.5, jnp.float32)
    three_half = jnp.full((16,), 1.5, jnp.float32)
    magic = jnp.full((16,), 0x5F3759DF, jnp.int32)
    ones16 = jnp.full((16,), 1.0, jnp.float32)

    def zero_acct():
        def z4(j, carry):
            for u in range(UNROLL):
                acct[pl.ds((j * UNROLL + u) * 16, 16)] = jnp.zeros(
                    (16,), jnp.float32)
            return carry
        lax.fori_loop(0, FW // (16 * UNROLL), z4, 0)

    def exchange_and_gather_partials():
        pltpu.sync_copy(acct, psh.at[s])
        plsc.subcore_barrier()
        cps = [pltpu.async_copy(psh.at[t, fsl], tmp16.at[t], rsem)
               for t in range(NS)]
        for cp in cps:
            cp.wait()
        plsc.subcore_barrier()

    def reduce_into_tmpb(base_buf):
        def red(i, carry):
            ix = pl.ds(i * 16, 16)
            acc = base_buf[ix]
            for t in range(NS):
                acc = acc + tmp16[t, ix]
            tmpb[ix] = acc
            return carry
        lax.fori_loop(0, VE, red, 0)

    zero_acct()

    def deg_blk(j, carry):
        for u in range(UNROLL):
            ix = pl.ds((j * UNROLL + u) * 16, 16)
            cx = colv[ix] * 2
            plsc.addupdate_scatter(acct, [cx], ones16)
            plsc.addupdate_scatter(acct, [cx + 1], ones16)
        return carry

    lax.fori_loop(0, NBLK, deg_blk, 0)
    exchange_and_gather_partials()
    zcp.wait()

    def newton(i, carry):
        ix = pl.ds(i * 16, 16)
        d16 = tmp16[0, ix] + 1.0
        for t in range(1, NS):
            d16 = d16 + tmp16[t, ix]
        h = d16 * half
        yi = magic - lax.shift_right_logical(plsc.bitcast(d16, jnp.int32), 1)
        y = plsc.bitcast(yi, jnp.float32)
        y = y * (three_half - h * y * y)
        y = y * (three_half - h * y * y)
        y = y * (three_half - h * y * y)
        disb[ix] = y
        dinvb[ix] = y * y
        zb[ix] = y * zb[ix]
        return carry

    lax.fori_loop(0, VE, newton, 0)
    pltpu.sync_copy(zb, tsh.at[fsl])
    plsc.subcore_barrier()
    pltpu.sync_copy(tsh, tbl)

    def hop():
        zero_acct()

        def blk(j, carry):
            for u in range(UNROLL):
                ix = pl.ds((j * UNROLL + u) * 16, 16)
                rx = rowv[ix] * 2
                cx = colv[ix] * 2
                v0 = plsc.load_gather(tbl, [rx])
                plsc.addupdate_scatter(acct, [cx], v0)
                v1 = plsc.load_gather(tbl, [rx + 1])
                plsc.addupdate_scatter(acct, [cx + 1], v1)
            return carry

        lax.fori_loop(0, NBLK, blk, 0)
        exchange_and_gather_partials()

    hop()
    reduce_into_tmpb(zb)

    def scale_w(i, carry):
        ix = pl.ds(i * 16, 16)
        w16 = tmpb[ix] * dinvb[ix]
        tmpb[ix] = w16
        zb[ix] = w16
        return carry

    lax.fori_loop(0, VE, scale_w, 0)
    pltpu.sync_copy(tmpb, tsh.at[fsl])
    plsc.subcore_barrier()
    pltpu.sync_copy(tsh, tbl)

    hop()
    reduce_into_tmpb(zb)

    def scale_out(i, carry):
        ix = pl.ds(i * 16, 16)
        tmpb[ix] = tmpb[ix] * disb[ix]
        return carry

    lax.fori_loop(0, VE, scale_out, 0)
    pltpu.sync_copy(tmpb, out_hbm.at[c, fsl])


def _tc_z(x_ref, w_ref, z_ref):
    z_ref[...] = jnp.dot(x_ref[...], w_ref[...],
                         preferred_element_type=jnp.float32)


def _tc_final(h_ref, b_ref, out_ref):
    o = h_ref[...] + b_ref[...]
    m = jnp.max(o, axis=1, keepdims=True)
    e = jnp.exp(o - m)
    lse = jnp.log(jnp.sum(e, axis=1, keepdims=True))
    out_ref[...] = o - m - lse


def kernel(x, edge_index, W, b):
    f32 = jnp.float32
    rows = edge_index[0]
    cols = edge_index[1]
    pad = E_PAD - E
    rows2 = jnp.concatenate(
        [rows, jnp.zeros((pad,), jnp.int32)]).reshape(NS, EPT)
    cols2 = jnp.concatenate(
        [cols, jnp.full((pad,), DUMMY, jnp.int32)]).reshape(NS, EPT)

    x_pad = jnp.pad(x, ((0, N_PAD - N), (0, 0)))
    z4 = pl.pallas_call(
        _tc_z,
        out_shape=jax.ShapeDtypeStruct((N_PAD, D_OUT), f32),
    )(x_pad, W)
    zs = jnp.transpose(z4.reshape(N_PAD, NC, DC), (1, 0, 2)).reshape(NC, FW)

    h = _sgconv_sc(zs, rows2, cols2)
    h4 = jnp.transpose(h.reshape(NC, N_PAD, DC), (1, 0, 2)).reshape(
        N_PAD, D_OUT)

    out = pl.pallas_call(
        _tc_final,
        out_shape=jax.ShapeDtypeStruct((N_PAD, D_OUT), f32),
    )(h4, b)
    return out[:N]


# speedup vs baseline: 1.4394x; 1.0592x over previous
import functools

import jax
import jax.numpy as jnp
from jax import lax
from jax.experimental import pallas as pl
from jax.experimental.pallas import tpu as pltpu
from jax.experimental.pallas import tpu_sc as plsc

N = 10000
E = 320000
D_IN = 128
D_OUT = 4

NC = 2
DC = 2
NS = 16
UNROLL = 5
EPT = E // NS
NBLK = EPT // (16 * UNROLL)
N_PAD = 10240
RPB = N_PAD // NS
FPB = RPB * DC
VE = FPB // 16
FW = N_PAD * DC

_mesh = plsc.VectorSubcoreMesh(
    core_axis_name="c", subcore_axis_name="s", num_cores=NC, num_subcores=NS
)


@functools.partial(
    pl.kernel,
    out_type=jax.ShapeDtypeStruct((NC, FW), jnp.float32),
    mesh=_mesh,
    scratch_types=[
        pltpu.VMEM((EPT,), jnp.int32),
        pltpu.VMEM((EPT,), jnp.int32),
        pltpu.VMEM((FW,), jnp.float32),
        pltpu.VMEM((FW,), jnp.float32),
        pltpu.VMEM((FPB,), jnp.float32),
        pltpu.VMEM((FPB,), jnp.float32),
        pltpu.VMEM((FPB,), jnp.float32),
        pltpu.VMEM((FPB,), jnp.float32),
        pltpu.VMEM((NS, FPB), jnp.float32),
        pltpu.VMEM_SHARED((NS, FW), jnp.float32),
        pltpu.VMEM_SHARED((FW,), jnp.float32),
        pltpu.SemaphoreType.DMA,
        pltpu.SemaphoreType.DMA,
    ],
    compiler_params=pltpu.CompilerParams(
        use_tc_tiling_on_sc=False, needs_layout_passes=False),
)
def _sgconv_sc(z_hbm, rows_hbm, cols_hbm, out_hbm,
               rowv, colv, tbl, acct, zb, disb, dinvb, tmpb, tmp16,
               psh, tsh, zsem, rsem):
    c = lax.axis_index("c")
    s = lax.axis_index("s")
    fsl = pl.ds(s * FPB, FPB)

    zcp = pltpu.async_copy(z_hbm.at[c, fsl], zb, zsem)
    pltpu.sync_copy(rows_hbm.at[s], rowv)
    pltpu.sync_copy(cols_hbm.at[s], colv)

    half = jnp.full((16,), 0.5, jnp.float32)
    three_half = jnp.full((16,), 1.5, jnp.float32)
    magic = jnp.full((16,), 0x5F3759DF, jnp.int32)
    ones16 = jnp.full((16,), 1.0, jnp.float32)

    def zero_acct():
        def z4(j, carry):
            for u in range(UNROLL):
                acct[pl.ds((j * UNROLL + u) * 16, 16)] = jnp.zeros(
                    (16,), jnp.float32)
            return carry
        lax.fori_loop(0, FW // (16 * UNROLL), z4, 0)

    def exchange_and_gather_partials():
        pltpu.sync_copy(acct, psh.at[s])
        plsc.subcore_barrier()
        cps = [pltpu.async_copy(psh.at[t, fsl], tmp16.at[t], rsem)
               for t in range(NS)]
        for cp in cps:
            cp.wait()
        plsc.subcore_barrier()

    def reduce_into_tmpb(base_buf):
        def red(i, carry):
            ix = pl.ds(i * 16, 16)
            acc = base_buf[ix]
            for t in range(NS):
                acc = acc + tmp16[t, ix]
            tmpb[ix] = acc
            return carry
        lax.fori_loop(0, VE, red, 0)

    zero_acct()

    def deg_blk(j, carry):
        for u in range(UNROLL):
            ix = pl.ds((j * UNROLL + u) * 16, 16)
            cx = colv[ix] * 2
            plsc.addupdate_scatter(acct, [cx], ones16)
            plsc.addupdate_scatter(acct, [cx + 1], ones16)
        return carry

    lax.fori_loop(0, NBLK, deg_blk, 0)
    exchange_and_gather_partials()
    zcp.wait()

    def newton(i, carry):
        ix = pl.ds(i * 16, 16)
        d16 = tmp16[0, ix] + 1.0
        for t in range(1, NS):
            d16 = d16 + tmp16[t, ix]
        h = d16 * half
        yi = magic - lax.shift_right_logical(plsc.bitcast(d16, jnp.int32), 1)
        y = plsc.bitcast(yi, jnp.float32)
        y = y * (three_half - h * y * y)
        y = y * (three_half - h * y * y)
        y = y * (three_half - h * y * y)
        disb[ix] = y
        dinvb[ix] = y * y
        zb[ix] = y * zb[ix]
        return carry

    lax.fori_loop(0, VE, newton, 0)
    pltpu.sync_copy(zb, tsh.at[fsl])
    plsc.subcore_barrier()
    pltpu.sync_copy(tsh, tbl)

    def hop():
        zero_acct()

        def blk(j, carry):
            for u in range(UNROLL):
                ix = pl.ds((j * UNROLL + u) * 16, 16)
                rx = rowv[ix] * 2
                cx = colv[ix] * 2
                v0 = plsc.load_gather(tbl, [rx])
                plsc.addupdate_scatter(acct, [cx], v0)
                v1 = plsc.load_gather(tbl, [rx + 1])
                plsc.addupdate_scatter(acct, [cx + 1], v1)
            return carry

        lax.fori_loop(0, NBLK, blk, 0)
        exchange_and_gather_partials()

    hop()
    reduce_into_tmpb(zb)

    def scale_w(i, carry):
        ix = pl.ds(i * 16, 16)
        w16 = tmpb[ix] * dinvb[ix]
        tmpb[ix] = w16
        zb[ix] = w16
        return carry

    lax.fori_loop(0, VE, scale_w, 0)
    pltpu.sync_copy(tmpb, tsh.at[fsl])
    plsc.subcore_barrier()
    pltpu.sync_copy(tsh, tbl)

    hop()
    reduce_into_tmpb(zb)

    def scale_out(i, carry):
        ix = pl.ds(i * 16, 16)
        tmpb[ix] = tmpb[ix] * disb[ix]
        return carry

    lax.fori_loop(0, VE, scale_out, 0)
    pltpu.sync_copy(tmpb, out_hbm.at[c, fsl])


def _tc_z(x_ref, w_ref, z_ref):
    z_ref[:N, :] = jnp.dot(x_ref[...], w_ref[...],
                           preferred_element_type=jnp.float32)
    z_ref[N:, :] = jnp.zeros((N_PAD - N, D_OUT), jnp.float32)


def _tc_final(h_ref, b_ref, out_ref):
    o = h_ref[...] + b_ref[...]
    m = jnp.max(o, axis=1, keepdims=True)
    e = jnp.exp(o - m)
    lse = jnp.log(jnp.sum(e, axis=1, keepdims=True))
    out_ref[...] = o - m - lse


def kernel(x, edge_index, W, b):
    f32 = jnp.float32
    rows2 = edge_index[0].reshape(NS, EPT)
    cols2 = edge_index[1].reshape(NS, EPT)

    z4 = pl.pallas_call(
        _tc_z,
        out_shape=jax.ShapeDtypeStruct((N_PAD, D_OUT), f32),
    )(x, W)
    zs = jnp.transpose(z4.reshape(N_PAD, NC, DC), (1, 0, 2)).reshape(NC, FW)

    h = _sgconv_sc(zs, rows2, cols2)
    h4 = jnp.transpose(h.reshape(NC, N_PAD, DC), (1, 0, 2)).reshape(
        N_PAD, D_OUT)

    out = pl.pallas_call(
        _tc_final,
        out_shape=jax.ShapeDtypeStruct((N_PAD, D_OUT), f32),
    )(h4, b)
    return out[:N]


# speedup vs baseline: 1.4579x; 1.0128x over previous
import functools

import jax
import jax.numpy as jnp
from jax import lax
from jax.experimental import pallas as pl
from jax.experimental.pallas import tpu as pltpu
from jax.experimental.pallas import tpu_sc as plsc

N = 10000
E = 320000
D_IN = 128
D_OUT = 4

NC = 2
DC = 2
NS = 16
UNROLL = 5
EPT = E // NS
NBLK = EPT // (16 * UNROLL)
N_PAD = 10240
RPB = N_PAD // NS
FPB = RPB * DC
VE = FPB // 16
FW = N_PAD * DC

_mesh = plsc.VectorSubcoreMesh(
    core_axis_name="c", subcore_axis_name="s", num_cores=NC, num_subcores=NS
)


@functools.partial(
    pl.kernel,
    out_type=jax.ShapeDtypeStruct((NC, FW), jnp.float32),
    mesh=_mesh,
    scratch_types=[
        pltpu.VMEM((EPT,), jnp.int32),
        pltpu.VMEM((EPT,), jnp.int32),
        pltpu.VMEM((FW,), jnp.float32),
        pltpu.VMEM((FW,), jnp.float32),
        pltpu.VMEM((FPB,), jnp.float32),
        pltpu.VMEM((FPB,), jnp.float32),
        pltpu.VMEM((FPB,), jnp.float32),
        pltpu.VMEM((FPB,), jnp.float32),
        pltpu.VMEM((NS, FPB), jnp.float32),
        pltpu.VMEM_SHARED((NS, FW), jnp.float32),
        pltpu.VMEM_SHARED((FW,), jnp.float32),
        pltpu.SemaphoreType.DMA,
        pltpu.SemaphoreType.DMA,
    ],
    compiler_params=pltpu.CompilerParams(
        use_tc_tiling_on_sc=False, needs_layout_passes=False),
)
def _sgconv_sc(z_hbm, rows_hbm, cols_hbm, out_hbm,
               rowv, colv, tbl, acct, zb, disb, dinvb, tmpb, tmp16,
               psh, tsh, zsem, rsem):
    c = lax.axis_index("c")
    s = lax.axis_index("s")
    fsl = pl.ds(s * FPB, FPB)

    zcp = pltpu.async_copy(z_hbm.at[c, fsl], zb, zsem)
    pltpu.sync_copy(rows_hbm.at[s], rowv)
    pltpu.sync_copy(cols_hbm.at[s], colv)

    half = jnp.full((16,), 0.5, jnp.float32)
    three_half = jnp.full((16,), 1.5, jnp.float32)
    magic = jnp.full((16,), 0x5F3759DF, jnp.int32)
    ones16 = jnp.full((16,), 1.0, jnp.float32)

    def zero_acct():
        def z4(j, carry):
            for u in range(UNROLL):
                acct[pl.ds((j * UNROLL + u) * 16, 16)] = jnp.zeros(
                    (16,), jnp.float32)
            return carry
        lax.fori_loop(0, FW // (16 * UNROLL), z4, 0)

    def exchange_and_gather_partials():
        pltpu.sync_copy(acct, psh.at[s])
        plsc.subcore_barrier()
        cps = [pltpu.async_copy(psh.at[t, fsl], tmp16.at[t], rsem)
               for t in range(NS)]
        for cp in cps:
            cp.wait()
        plsc.subcore_barrier()

    def reduce_into_tmpb(base_buf):
        def red(i, carry):
            ix = pl.ds(i * 16, 16)
            acc = base_buf[ix]
            for t in range(NS):
                acc = acc + tmp16[t, ix]
            tmpb[ix] = acc
            return carry
        lax.fori_loop(0, VE, red, 0)

    zero_acct()

    def deg_blk(j, carry):
        for u in range(UNROLL):
            ix = pl.ds((j * UNROLL + u) * 16, 16)
            cx = colv[ix]
            plsc.addupdate_scatter(acct, [cx], ones16)
            plsc.addupdate_scatter(acct, [cx + 1], ones16)
        return carry

    lax.fori_loop(0, NBLK, deg_blk, 0)
    exchange_and_gather_partials()
    zcp.wait()

    def newton(i, carry):
        ix = pl.ds(i * 16, 16)
        d16 = tmp16[0, ix] + 1.0
        for t in range(1, NS):
            d16 = d16 + tmp16[t, ix]
        h = d16 * half
        yi = magic - lax.shift_right_logical(plsc.bitcast(d16, jnp.int32), 1)
        y = plsc.bitcast(yi, jnp.float32)
        y = y * (three_half - h * y * y)
        y = y * (three_half - h * y * y)
        y = y * (three_half - h * y * y)
        disb[ix] = y
        dinvb[ix] = y * y
        zb[ix] = y * zb[ix]
        return carry

    lax.fori_loop(0, VE, newton, 0)
    pltpu.sync_copy(zb, tsh.at[fsl])
    plsc.subcore_barrier()
    pltpu.sync_copy(tsh, tbl)

    def hop():
        zero_acct()

        def blk(j, carry):
            for u in range(UNROLL):
                ix = pl.ds((j * UNROLL + u) * 16, 16)
                rx = rowv[ix]
                cx = colv[ix]
                v0 = plsc.load_gather(tbl, [rx])
                plsc.addupdate_scatter(acct, [cx], v0)
                v1 = plsc.load_gather(tbl, [rx + 1])
                plsc.addupdate_scatter(acct, [cx + 1], v1)
            return carry

        lax.fori_loop(0, NBLK, blk, 0)
        exchange_and_gather_partials()

    hop()
    reduce_into_tmpb(zb)

    def scale_w(i, carry):
        ix = pl.ds(i * 16, 16)
        w16 = tmpb[ix] * dinvb[ix]
        tmpb[ix] = w16
        zb[ix] = w16
        return carry

    lax.fori_loop(0, VE, scale_w, 0)
    pltpu.sync_copy(tmpb, tsh.at[fsl])
    plsc.subcore_barrier()
    pltpu.sync_copy(tsh, tbl)

    hop()
    reduce_into_tmpb(zb)

    def scale_out(i, carry):
        ix = pl.ds(i * 16, 16)
        tmpb[ix] = tmpb[ix] * disb[ix]
        return carry

    lax.fori_loop(0, VE, scale_out, 0)
    pltpu.sync_copy(tmpb, out_hbm.at[c, fsl])


def _tc_z(x_ref, w_ref, z_ref):
    z_ref[:N, :] = jnp.dot(x_ref[...], w_ref[...],
                           preferred_element_type=jnp.float32)
    z_ref[N:, :] = jnp.zeros((N_PAD - N, D_OUT), jnp.float32)


def _tc_final(h_ref, b_ref, out_ref):
    o = h_ref[...] + b_ref[...]
    m = jnp.max(o, axis=1, keepdims=True)
    e = jnp.exp(o - m)
    lse = jnp.log(jnp.sum(e, axis=1, keepdims=True))
    out_ref[...] = o - m - lse


def kernel(x, edge_index, W, b):
    f32 = jnp.float32
    rows2 = (edge_index[0] * 2).reshape(NS, EPT)
    cols2 = (edge_index[1] * 2).reshape(NS, EPT)

    z4 = pl.pallas_call(
        _tc_z,
        out_shape=jax.ShapeDtypeStruct((N_PAD, D_OUT), f32),
    )(x, W)
    zs = jnp.transpose(z4.reshape(N_PAD, NC, DC), (1, 0, 2)).reshape(NC, FW)

    h = _sgconv_sc(zs, rows2, cols2)
    h4 = jnp.transpose(h.reshape(NC, N_PAD, DC), (1, 0, 2)).reshape(
        N_PAD, D_OUT)

    out = pl.pallas_call(
        _tc_final,
        out_shape=jax.ShapeDtypeStruct((N_PAD, D_OUT), f32),
    )(h4, b)
    return out[:N]
